# Initial kernel scaffold; baseline (speedup 1.0000x reference)
import jax
import jax.numpy as jnp
from jax.experimental import pallas as pl


def kernel(node_types, node_labels, edge_types, edge_labels, edge_index, emb, W_self0, W_src0, W_edge0, b0, W_self1, W_src1, W_edge1, b1, W_fc, b_fc):
    raise NotImplementedError("write your pallas kernel here")


# speedup vs baseline: 7.4296x; 7.4296x over previous
import functools

import jax
import jax.numpy as jnp
from jax import lax
from jax.experimental import pallas as pl
from jax.experimental.pallas import tpu as pltpu
from jax.experimental.pallas import tpu_sc as plsc

N = 50000
E = 800000
H = 64
NCLS = 10

NC = 2
NS = 16
LANES = 16

CH = 128
KPG = 8
GRP = CH * KPG
GROUPS = 49
EPT = GROUPS * GRP
EPAD = NS * EPT
NACC = 50176
ZPT = NACC // NS
DPT = N // NS

NLPAD = 204800
NLPT = NLPAD // (NC * NS)

R = 2000
NBLK = N // R

_f32 = jnp.float32
_i32 = jnp.int32

_mesh = plsc.VectorSubcoreMesh(
    core_axis_name="c", subcore_axis_name="s", num_cores=NC, num_subcores=NS)
_sc_params = pltpu.CompilerParams(use_tc_tiling_on_sc=False)


def _zero_fill(buf, rows, w):
    def zb(i, carry):
        for c in range(w // LANES):
            buf[i, pl.ds(c * LANES, LANES)] = jnp.zeros((LANES,), _f32)
        return carry
    lax.fori_loop(0, rows, zb, 0)


def _zero_acc(acc, buf, sid, grp):
    zb0 = sid * ZPT
    off = 0
    while off < ZPT:
        sz = min(grp, ZPT - off)
        pltpu.sync_copy(buf.at[pl.ds(0, sz)], acc.at[pl.ds(zb0 + off, sz)])
        off += sz


def _drain_acc(acc, out_hbm, cid, sid):
    db = sid * DPT
    ob = cid * N + db
    for off, sz in ((0, 1024), (1024, 1024), (2048, 1024), (3072, DPT - 3072)):
        pltpu.sync_copy(acc.at[pl.ds(db + off, sz)],
                        out_hbm.at[pl.ds(ob + off, sz)])


@functools.partial(
    pl.kernel,
    out_type=jax.ShapeDtypeStruct((NLPAD, 16), _f32),
    mesh=_mesh,
    compiler_params=_sc_params,
    scratch_types=[
        pltpu.VMEM((NLPT,), _i32),
        pltpu.VMEM((1280, 16), _f32),
        pltpu.SemaphoreType.DMA,
    ],
)
def _sc_node_embed(emb_hbm, nl_hbm, out_hbm, idxv, rows, sem):
    cid = lax.axis_index("c")
    sid = lax.axis_index("s")
    wid = sid * NC + cid
    base = wid * NLPT
    pltpu.sync_copy(nl_hbm.at[pl.ds(base, NLPT)], idxv)

    def gbody(g, carry):
        gb = g * 1280
        cps = []
        for k in range(10):
            cps.append(pltpu.async_copy(
                emb_hbm.at[idxv.at[pl.ds(gb + k * CH, CH)]],
                rows.at[pl.ds(k * CH, CH)], sem))
        for cp in cps:
            cp.wait()
        pltpu.sync_copy(rows, out_hbm.at[pl.ds(base + gb, 1280)])
        return carry
    lax.fori_loop(0, NLPT // 1280, gbody, 0)


def _make_segsum(w, kpg):
    grp = kpg * CH
    groups = EPT // grp

    @functools.partial(
        pl.kernel,
        out_type=jax.ShapeDtypeStruct((NC * N, w), _f32),
        mesh=_mesh,
        compiler_params=_sc_params,
        scratch_types=[
            pltpu.VMEM_SHARED((NACC, w), _f32),
            pltpu.VMEM((grp,), _i32),
            pltpu.VMEM((kpg, CH), _i32),
            pltpu.VMEM((grp, w), _f32),
            pltpu.SemaphoreType.DMA,
        ],
    )
    def k(tbl_hbm, src_hbm, dst_hbm, out_hbm, acc, srcv, dstv, rows, sem):
        cid = lax.axis_index("c")
        sid = lax.axis_index("s")
        _zero_fill(rows, grp, w)
        _zero_acc(acc, rows, sid, grp)
        plsc.subcore_barrier()

        tb = sid * EPT
        itb = cid * EPAD + tb
        rtb = tb // CH

        def gbody(g, carry):
            gb = g * grp
            pltpu.sync_copy(src_hbm.at[pl.ds(itb + gb, grp)], srcv)
            pltpu.sync_copy(dst_hbm.at[pl.ds(rtb + g * kpg, kpg)], dstv)
            cps = []
            for k_ in range(kpg):
                cps.append(pltpu.async_copy(
                    tbl_hbm.at[srcv.at[pl.ds(k_ * CH, CH)]],
                    rows.at[pl.ds(k_ * CH, CH)], sem))
            for k_ in range(kpg):
                cps[k_].wait()
                pltpu.sync_copy(rows.at[pl.ds(k_ * CH, CH)],
                                acc.at[dstv.at[k_]], add=True)
            return carry
        lax.fori_loop(0, groups, gbody, 0)
        plsc.subcore_barrier()
        _drain_acc(acc, out_hbm, cid, sid)
    return k


_sc_segsum16 = _make_segsum(16, 8)
_sc_segsum32 = _make_segsum(32, 4)


EA_KPG = 4
EA_GRP = EA_KPG * CH
EA_GROUPS = EPT // EA_GRP
ET_SPLIT = 49


@functools.partial(
    pl.kernel,
    out_type=(jax.ShapeDtypeStruct((NC * N, 16), _f32),
              jax.ShapeDtypeStruct((NC * N, 8), _f32)),
    mesh=_mesh,
    compiler_params=_sc_params,
    scratch_types=[
        pltpu.VMEM_SHARED((NACC, 16), _f32),
        pltpu.VMEM_SHARED((NACC, 8), _f32),
        pltpu.VMEM((EA_GRP,), _i32),
        pltpu.VMEM((EA_KPG, CH), _i32),
        pltpu.VMEM((EA_GRP, 16), _f32),
        pltpu.VMEM((EA_GRP, 8), _f32),
        pltpu.SemaphoreType.DMA,
    ],
)
def _sc_edge_aggs(emb_hbm, el_hbm, dst_hbm, et_hbm, oute_hbm, outt_hbm,
                  acc_e, acc_t, elv, dstv, erows, etv, sem):
    cid = lax.axis_index("c")
    sid = lax.axis_index("s")
    _zero_fill(erows, EA_GRP, 16)
    _zero_acc(acc_e, erows, sid, EA_GRP)

    def z8(i, carry):
        etv[i, :] = jnp.zeros((8,), _f32)
        return carry
    lax.fori_loop(0, EA_GRP, z8, 0)
    _zero_acc(acc_t, etv, sid, EA_GRP)
    plsc.subcore_barrier()

    tb = sid * EPT
    itb = cid * EPAD + tb
    rtb = tb // CH

    def gbody(g, carry):
        gb = g * EA_GRP
        pltpu.sync_copy(el_hbm.at[pl.ds(itb + gb, EA_GRP)], elv)
        pltpu.sync_copy(dst_hbm.at[pl.ds(rtb + g * EA_KPG, EA_KPG)], dstv)
        cps = []
        for k_ in range(EA_KPG):
            cps.append(pltpu.async_copy(
                emb_hbm.at[elv.at[pl.ds(k_ * CH, CH)]],
                erows.at[pl.ds(k_ * CH, CH)], sem))
        mine = jnp.where(cid == 0, g < ET_SPLIT, g >= ET_SPLIT)

        @pl.when(mine)
        def _():
            for k_ in range(EA_KPG):
                r0 = tb + gb + k_ * CH

                @pl.when(r0 < E)
                def _chunk(k_=k_, r0=r0):
                    pltpu.sync_copy(et_hbm.at[pl.ds(r0, CH)],
                                    etv.at[pl.ds(k_ * CH, CH)])
                    pltpu.sync_copy(etv.at[pl.ds(k_ * CH, CH)],
                                    acc_t.at[dstv.at[k_]], add=True)

        for k_ in range(EA_KPG):
            cps[k_].wait()
            pltpu.sync_copy(erows.at[pl.ds(k_ * CH, CH)],
                            acc_e.at[dstv.at[k_]], add=True)
        return carry
    lax.fori_loop(0, EA_GROUPS, gbody, 0)
    plsc.subcore_barrier()
    _drain_acc(acc_e, oute_hbm, cid, sid)
    _drain_acc(acc_t, outt_hbm, cid, sid)


def _tc_build_nf(nt_ref, ne_ref, out_ref):
    nt = nt_ref[...]
    ne = ne_ref[...]
    h0 = jnp.concatenate([nt, ne[:, 0:5], ne[:, 16:19]], axis=1)
    h1 = jnp.concatenate([ne[:, 19:21], ne[:, 32:37], ne[:, 48:53],
                          jnp.zeros((R, 4), _f32)], axis=1)
    out_ref[0] = h0
    out_ref[1] = h1


def _tc_layer0(nf_ref, a_ref, ae_ref, at_ref, wb_ref, b_ref, out_ref):
    x = jnp.concatenate([nf_ref[0], nf_ref[1], a_ref[0], a_ref[1],
                         ae_ref[0], ae_ref[1], at_ref[0], at_ref[1]], axis=1)
    h = jnp.dot(x, wb_ref[...], preferred_element_type=_f32) + b_ref[...]
    h = jnp.maximum(h, 0.0)
    out_ref[0] = h[:, :32]
    out_ref[1] = h[:, 32:]


def _tc_layer1(h_ref, a_ref, ae_ref, at_ref, wb_ref, b_ref, out_ref):
    x = jnp.concatenate([h_ref[0], h_ref[1], a_ref[0], a_ref[1],
                         ae_ref[0], ae_ref[1], at_ref[0], at_ref[1]], axis=1)
    h = jnp.dot(x, wb_ref[...], preferred_element_type=_f32) + b_ref[...]
    h = jnp.maximum(h, 0.0)
    out_ref[pl.ds(pl.program_id(0), 1), :] = jnp.sum(h, axis=0, keepdims=True)


def _tc_final(ps_ref, wfc_ref, bfc_ref, out_ref):
    s = jnp.sum(ps_ref[...], axis=0, keepdims=True)
    out_ref[...] = jnp.dot(s, wfc_ref[...], preferred_element_type=_f32) \
        + bfc_ref[...]


def _pad_rows(w, rows):
    return jnp.concatenate(
        [w, jnp.zeros((rows - w.shape[0], w.shape[1]), _f32)], axis=0)


def kernel(node_types, node_labels, edge_types, edge_labels, edge_index, emb,
           W_self0, W_src0, W_edge0, b0, W_self1, W_src1, W_edge1, b1,
           W_fc, b_fc):
    emb16 = jnp.zeros((emb.shape[0], 16), _f32).at[:, :5].set(emb)
    nl_flat = jnp.concatenate(
        [node_labels.reshape(-1).astype(_i32),
         jnp.zeros((NLPAD - 4 * N,), _i32)])

    src = edge_index[0].astype(_i32)
    dst = edge_index[1].astype(_i32)
    padn = EPAD - E
    srcp = jnp.concatenate([src, jnp.zeros((padn,), _i32)])
    src2 = jnp.concatenate([srcp, srcp + N])
    dstp = jnp.concatenate([dst, jnp.full((padn,), N, _i32)])
    dst2d = dstp.reshape(EPAD // CH, CH)
    el0 = edge_labels[:, 0].astype(_i32)
    el1 = edge_labels[:, 1].astype(_i32)
    el2 = jnp.concatenate([
        jnp.concatenate([el0, jnp.zeros((padn,), _i32)]),
        jnp.concatenate([el1, jnp.zeros((padn,), _i32)])])

    wb0 = jnp.concatenate([
        _pad_rows(W_self0, 32), _pad_rows(W_src0, 32),
        _pad_rows(W_edge0[0:5], 16), _pad_rows(W_edge0[5:10], 16),
        W_edge0[10:18], W_edge0[10:18]], axis=0)
    wb1 = jnp.concatenate([
        W_self1, W_src1,
        _pad_rows(W_edge1[0:5], 16), _pad_rows(W_edge1[5:10], 16),
        W_edge1[10:18], W_edge1[10:18]], axis=0)
    b0r = b0.reshape(1, H)
    b1r = b1.reshape(1, H)
    bfcr = b_fc.reshape(1, NCLS)

    ne_pad = _sc_node_embed(emb16, nl_flat)
    ne64 = ne_pad[:4 * N].reshape(N, 64)

    nf = pl.pallas_call(
        _tc_build_nf,
        grid=(NBLK,),
        in_specs=[
            pl.BlockSpec((R, 8), lambda i: (i, 0)),
            pl.BlockSpec((R, 64), lambda i: (i, 0)),
        ],
        out_specs=pl.BlockSpec((2, R, 16), lambda i: (0, i, 0)),
        out_shape=jax.ShapeDtypeStruct((2, N, 16), _f32),
    )(node_types, ne64)

    a0 = _sc_segsum16(nf.reshape(NC * N, 16), src2, dst2d)
    agg_e, agg_t = _sc_edge_aggs(emb16, el2, dst2d, edge_types)
    a0 = a0.reshape(2, N, 16)
    agg_e = agg_e.reshape(2, N, 16)
    agg_t = agg_t.reshape(2, N, 8)

    h1 = pl.pallas_call(
        _tc_layer0,
        grid=(NBLK,),
        in_specs=[
            pl.BlockSpec((2, R, 16), lambda i: (0, i, 0)),
            pl.BlockSpec((2, R, 16), lambda i: (0, i, 0)),
            pl.BlockSpec((2, R, 16), lambda i: (0, i, 0)),
            pl.BlockSpec((2, R, 8), lambda i: (0, i, 0)),
            pl.BlockSpec((112, H), lambda i: (0, 0)),
            pl.BlockSpec((1, H), lambda i: (0, 0)),
        ],
        out_specs=pl.BlockSpec((2, R, 32), lambda i: (0, i, 0)),
        out_shape=jax.ShapeDtypeStruct((2, N, 32), _f32),
    )(nf, a0, agg_e, agg_t, wb0, b0r)

    a1 = _sc_segsum32(h1.reshape(NC * N, 32), src2, dst2d).reshape(2, N, 32)

    psums = pl.pallas_call(
        _tc_layer1,
        grid=(NBLK,),
        in_specs=[
            pl.BlockSpec((2, R, 32), lambda i: (0, i, 0)),
            pl.BlockSpec((2, R, 32), lambda i: (0, i, 0)),
            pl.BlockSpec((2, R, 16), lambda i: (0, i, 0)),
            pl.BlockSpec((2, R, 8), lambda i: (0, i, 0)),
            pl.BlockSpec((176, H), lambda i: (0, 0)),
            pl.BlockSpec((1, H), lambda i: (0, 0)),
        ],
        out_specs=pl.BlockSpec((NBLK, H), lambda i: (0, 0)),
        out_shape=jax.ShapeDtypeStruct((NBLK, H), _f32),
    )(h1, a1, agg_e, agg_t, wb1, b1r)

    out = pl.pallas_call(
        _tc_final,
        out_shape=jax.ShapeDtypeStruct((1, NCLS), _f32),
    )(psums, W_fc, bfcr)
    return out


# speedup vs baseline: 7.5106x; 1.0109x over previous
import functools

import jax
import jax.numpy as jnp
from jax import lax
from jax.experimental import pallas as pl
from jax.experimental.pallas import tpu as pltpu
from jax.experimental.pallas import tpu_sc as plsc

N = 50000
E = 800000
H = 64
NCLS = 10

NC = 2
NS = 16
LANES = 16

CH = 128
KPG = 8
GRP = CH * KPG
GROUPS = 49
EPT = GROUPS * GRP
EPAD = NS * EPT
NACC = 50176
ZPT = NACC // NS
DPT = N // NS

NLPAD = 204800
NLPT = NLPAD // (NC * NS)

R = 2000
NBLK = N // R

_f32 = jnp.float32
_i32 = jnp.int32

_mesh = plsc.VectorSubcoreMesh(
    core_axis_name="c", subcore_axis_name="s", num_cores=NC, num_subcores=NS)
_sc_params = pltpu.CompilerParams(use_tc_tiling_on_sc=False)


def _zero_fill(buf, rows, w):
    def zb(i, carry):
        for c in range(w // LANES):
            buf[i, pl.ds(c * LANES, LANES)] = jnp.zeros((LANES,), _f32)
        return carry
    lax.fori_loop(0, rows, zb, 0)


def _zero_acc(acc, buf, sid, grp):
    zb0 = sid * ZPT
    off = 0
    while off < ZPT:
        sz = min(grp, ZPT - off)
        pltpu.sync_copy(buf.at[pl.ds(0, sz)], acc.at[pl.ds(zb0 + off, sz)])
        off += sz


def _drain_acc(acc, out_hbm, cid, sid):
    db = sid * DPT
    ob = cid * N + db
    for off, sz in ((0, 1024), (1024, 1024), (2048, 1024), (3072, DPT - 3072)):
        pltpu.sync_copy(acc.at[pl.ds(db + off, sz)],
                        out_hbm.at[pl.ds(ob + off, sz)])


@functools.partial(
    pl.kernel,
    out_type=jax.ShapeDtypeStruct((NLPAD, 16), _f32),
    mesh=_mesh,
    compiler_params=_sc_params,
    scratch_types=[
        pltpu.VMEM((NLPT,), _i32),
        pltpu.VMEM((1280, 16), _f32),
        pltpu.SemaphoreType.DMA,
    ],
)
def _sc_node_embed(emb_hbm, nl_hbm, out_hbm, idxv, rows, sem):
    cid = lax.axis_index("c")
    sid = lax.axis_index("s")
    wid = sid * NC + cid
    base = wid * NLPT
    pltpu.sync_copy(nl_hbm.at[pl.ds(base, NLPT)], idxv)

    def gbody(g, carry):
        gb = g * 1280
        cps = []
        for k in range(10):
            cps.append(pltpu.async_copy(
                emb_hbm.at[idxv.at[pl.ds(gb + k * CH, CH)]],
                rows.at[pl.ds(k * CH, CH)], sem))
        for cp in cps:
            cp.wait()
        pltpu.sync_copy(rows, out_hbm.at[pl.ds(base + gb, 1280)])
        return carry
    lax.fori_loop(0, NLPT // 1280, gbody, 0)


def _make_segsum(w, kpg):
    grp = kpg * CH
    groups = EPT // grp

    @functools.partial(
        pl.kernel,
        out_type=jax.ShapeDtypeStruct((NC * N, w), _f32),
        mesh=_mesh,
        compiler_params=_sc_params,
        scratch_types=[
            pltpu.VMEM_SHARED((NACC, w), _f32),
            pltpu.VMEM((grp,), _i32),
            pltpu.VMEM((kpg, CH), _i32),
            pltpu.VMEM((grp, w), _f32),
            pltpu.SemaphoreType.DMA,
            pltpu.SemaphoreType.DMA,
        ],
    )
    def k(tbl_hbm, src_hbm, dst_hbm, out_hbm, acc, srcv, dstv, rows, sem, ssem):
        cid = lax.axis_index("c")
        sid = lax.axis_index("s")
        _zero_fill(rows, grp, w)
        _zero_acc(acc, rows, sid, grp)
        plsc.subcore_barrier()

        tb = sid * EPT
        itb = cid * EPAD + tb
        rtb = tb // CH

        def gbody(g, carry):
            gb = g * grp
            pltpu.sync_copy(src_hbm.at[pl.ds(itb + gb, grp)], srcv)
            pltpu.sync_copy(dst_hbm.at[pl.ds(rtb + g * kpg, kpg)], dstv)
            cps = []
            for k_ in range(kpg):
                cps.append(pltpu.async_copy(
                    tbl_hbm.at[srcv.at[pl.ds(k_ * CH, CH)]],
                    rows.at[pl.ds(k_ * CH, CH)], sem))
            scs = []
            for k_ in range(kpg):
                cps[k_].wait()
                scs.append(pltpu.async_copy(
                    rows.at[pl.ds(k_ * CH, CH)],
                    acc.at[dstv.at[k_]], ssem, add=True))
            for sc in scs:
                sc.wait()
            return carry
        lax.fori_loop(0, groups, gbody, 0)
        plsc.subcore_barrier()
        _drain_acc(acc, out_hbm, cid, sid)
    return k


_sc_segsum32 = _make_segsum(32, 4)


S16_KPG = 8
S16_GRP = S16_KPG * CH
S16_GROUPS = EPT // S16_GRP
S16_SPLIT = 25


@functools.partial(
    pl.kernel,
    out_type=(jax.ShapeDtypeStruct((NC * N, 16), _f32),
              jax.ShapeDtypeStruct((NC * N, 8), _f32)),
    mesh=_mesh,
    compiler_params=_sc_params,
    scratch_types=[
        pltpu.VMEM_SHARED((NACC, 16), _f32),
        pltpu.VMEM_SHARED((NACC, 8), _f32),
        pltpu.VMEM((S16_GRP,), _i32),
        pltpu.VMEM((S16_KPG, CH), _i32),
        pltpu.VMEM((S16_GRP, 16), _f32),
        pltpu.VMEM((S16_GRP, 8), _f32),
        pltpu.SemaphoreType.DMA,
        pltpu.SemaphoreType.DMA,
    ],
)
def _sc_segsum16et(tbl_hbm, src_hbm, dst_hbm, et_hbm, out_hbm, outt_hbm,
                   acc, acc_t, srcv, dstv, rows, etv, sem, ssem):
    cid = lax.axis_index("c")
    sid = lax.axis_index("s")
    _zero_fill(rows, S16_GRP, 16)
    _zero_acc(acc, rows, sid, S16_GRP)

    def z8(i, carry):
        etv[i, :] = jnp.zeros((8,), _f32)
        return carry
    lax.fori_loop(0, S16_GRP, z8, 0)
    _zero_acc(acc_t, etv, sid, S16_GRP)
    plsc.subcore_barrier()

    tb = sid * EPT
    itb = cid * EPAD + tb
    rtb = tb // CH

    def gbody(g, carry):
        gb = g * S16_GRP
        pltpu.sync_copy(src_hbm.at[pl.ds(itb + gb, S16_GRP)], srcv)
        pltpu.sync_copy(dst_hbm.at[pl.ds(rtb + g * S16_KPG, S16_KPG)], dstv)
        cps = []
        for k_ in range(S16_KPG):
            cps.append(pltpu.async_copy(
                tbl_hbm.at[srcv.at[pl.ds(k_ * CH, CH)]],
                rows.at[pl.ds(k_ * CH, CH)], sem))
        mine = jnp.where(cid == 0, g < S16_SPLIT, g >= S16_SPLIT)
        scs = []

        @pl.when(mine)
        def _():
            for k_ in range(S16_KPG):
                r0 = tb + gb + k_ * CH

                @pl.when(r0 < E)
                def _chunk(k_=k_, r0=r0):
                    pltpu.sync_copy(et_hbm.at[pl.ds(r0, CH)],
                                    etv.at[pl.ds(k_ * CH, CH)])
                    pltpu.sync_copy(etv.at[pl.ds(k_ * CH, CH)],
                                    acc_t.at[dstv.at[k_]], add=True)

        for k_ in range(S16_KPG):
            cps[k_].wait()
            scs.append(pltpu.async_copy(
                rows.at[pl.ds(k_ * CH, CH)],
                acc.at[dstv.at[k_]], ssem, add=True))
        for sc in scs:
            sc.wait()
        return carry
    lax.fori_loop(0, S16_GROUPS, gbody, 0)
    plsc.subcore_barrier()
    _drain_acc(acc, out_hbm, cid, sid)
    _drain_acc(acc_t, outt_hbm, cid, sid)


EA_KPG = 8
EA_GRP = EA_KPG * CH
EA_GROUPS = EPT // EA_GRP


@functools.partial(
    pl.kernel,
    out_type=jax.ShapeDtypeStruct((NC * N, 16), _f32),
    mesh=_mesh,
    compiler_params=_sc_params,
    scratch_types=[
        pltpu.VMEM_SHARED((NACC, 16), _f32),
        pltpu.VMEM((EA_GRP,), _i32),
        pltpu.VMEM((EA_KPG, CH), _i32),
        pltpu.VMEM((EA_GRP, 16), _f32),
        pltpu.SemaphoreType.DMA,
        pltpu.SemaphoreType.DMA,
    ],
)
def _sc_edge_aggs(emb_hbm, el_hbm, dst_hbm, oute_hbm,
                  acc_e, elv, dstv, erows, sem, ssem):
    cid = lax.axis_index("c")
    sid = lax.axis_index("s")
    _zero_fill(erows, EA_GRP, 16)
    _zero_acc(acc_e, erows, sid, EA_GRP)
    plsc.subcore_barrier()

    tb = sid * EPT
    itb = cid * EPAD + tb
    rtb = tb // CH

    def gbody(g, carry):
        gb = g * EA_GRP
        pltpu.sync_copy(el_hbm.at[pl.ds(itb + gb, EA_GRP)], elv)
        pltpu.sync_copy(dst_hbm.at[pl.ds(rtb + g * EA_KPG, EA_KPG)], dstv)
        cps = []
        for k_ in range(EA_KPG):
            cps.append(pltpu.async_copy(
                emb_hbm.at[elv.at[pl.ds(k_ * CH, CH)]],
                erows.at[pl.ds(k_ * CH, CH)], sem))
        scs = []
        for k_ in range(EA_KPG):
            cps[k_].wait()
            scs.append(pltpu.async_copy(
                erows.at[pl.ds(k_ * CH, CH)],
                acc_e.at[dstv.at[k_]], ssem, add=True))
        for sc in scs:
            sc.wait()
        return carry
    lax.fori_loop(0, EA_GROUPS, gbody, 0)
    plsc.subcore_barrier()
    _drain_acc(acc_e, oute_hbm, cid, sid)


def _tc_build_nf(nt_ref, ne_ref, out_ref):
    nt = nt_ref[...]
    ne = ne_ref[...]
    h0 = jnp.concatenate([nt, ne[:, 0:5], ne[:, 16:19]], axis=1)
    h1 = jnp.concatenate([ne[:, 19:21], ne[:, 32:37], ne[:, 48:53],
                          jnp.zeros((R, 4), _f32)], axis=1)
    out_ref[0] = h0
    out_ref[1] = h1


def _tc_layer0(nf_ref, a_ref, ae_ref, at_ref, wb_ref, b_ref, out_ref):
    x = jnp.concatenate([nf_ref[0], nf_ref[1], a_ref[0], a_ref[1],
                         ae_ref[0], ae_ref[1], at_ref[0], at_ref[1]], axis=1)
    h = jnp.dot(x, wb_ref[...], preferred_element_type=_f32) + b_ref[...]
    h = jnp.maximum(h, 0.0)
    out_ref[0] = h[:, :32]
    out_ref[1] = h[:, 32:]


def _tc_layer1(h_ref, a_ref, ae_ref, at_ref, wb_ref, b_ref, out_ref):
    x = jnp.concatenate([h_ref[0], h_ref[1], a_ref[0], a_ref[1],
                         ae_ref[0], ae_ref[1], at_ref[0], at_ref[1]], axis=1)
    h = jnp.dot(x, wb_ref[...], preferred_element_type=_f32) + b_ref[...]
    h = jnp.maximum(h, 0.0)
    out_ref[pl.ds(pl.program_id(0), 1), :] = jnp.sum(h, axis=0, keepdims=True)


def _tc_final(ps_ref, wfc_ref, bfc_ref, out_ref):
    s = jnp.sum(ps_ref[...], axis=0, keepdims=True)
    out_ref[...] = jnp.dot(s, wfc_ref[...], preferred_element_type=_f32) \
        + bfc_ref[...]


def _pad_rows(w, rows):
    return jnp.concatenate(
        [w, jnp.zeros((rows - w.shape[0], w.shape[1]), _f32)], axis=0)


def kernel(node_types, node_labels, edge_types, edge_labels, edge_index, emb,
           W_self0, W_src0, W_edge0, b0, W_self1, W_src1, W_edge1, b1,
           W_fc, b_fc):
    emb16 = jnp.zeros((emb.shape[0], 16), _f32).at[:, :5].set(emb)
    nl_flat = jnp.concatenate(
        [node_labels.reshape(-1).astype(_i32),
         jnp.zeros((NLPAD - 4 * N,), _i32)])

    src = edge_index[0].astype(_i32)
    dst = edge_index[1].astype(_i32)
    padn = EPAD - E
    srcp = jnp.concatenate([src, jnp.zeros((padn,), _i32)])
    src2 = jnp.concatenate([srcp, srcp + N])
    dstp = jnp.concatenate([dst, jnp.full((padn,), N, _i32)])
    dst2d = dstp.reshape(EPAD // CH, CH)
    el0 = edge_labels[:, 0].astype(_i32)
    el1 = edge_labels[:, 1].astype(_i32)
    el2 = jnp.concatenate([
        jnp.concatenate([el0, jnp.zeros((padn,), _i32)]),
        jnp.concatenate([el1, jnp.zeros((padn,), _i32)])])

    wb0 = jnp.concatenate([
        _pad_rows(W_self0, 32), _pad_rows(W_src0, 32),
        _pad_rows(W_edge0[0:5], 16), _pad_rows(W_edge0[5:10], 16),
        W_edge0[10:18], W_edge0[10:18]], axis=0)
    wb1 = jnp.concatenate([
        W_self1, W_src1,
        _pad_rows(W_edge1[0:5], 16), _pad_rows(W_edge1[5:10], 16),
        W_edge1[10:18], W_edge1[10:18]], axis=0)
    b0r = b0.reshape(1, H)
    b1r = b1.reshape(1, H)
    bfcr = b_fc.reshape(1, NCLS)

    ne_pad = _sc_node_embed(emb16, nl_flat)
    ne64 = ne_pad[:4 * N].reshape(N, 64)

    nf = pl.pallas_call(
        _tc_build_nf,
        grid=(NBLK,),
        in_specs=[
            pl.BlockSpec((R, 8), lambda i: (i, 0)),
            pl.BlockSpec((R, 64), lambda i: (i, 0)),
        ],
        out_specs=pl.BlockSpec((2, R, 16), lambda i: (0, i, 0)),
        out_shape=jax.ShapeDtypeStruct((2, N, 16), _f32),
    )(node_types, ne64)

    a0, agg_t = _sc_segsum16et(nf.reshape(NC * N, 16), src2, dst2d, edge_types)
    agg_e = _sc_edge_aggs(emb16, el2, dst2d)
    a0 = a0.reshape(2, N, 16)
    agg_e = agg_e.reshape(2, N, 16)
    agg_t = agg_t.reshape(2, N, 8)

    h1 = pl.pallas_call(
        _tc_layer0,
        grid=(NBLK,),
        in_specs=[
            pl.BlockSpec((2, R, 16), lambda i: (0, i, 0)),
            pl.BlockSpec((2, R, 16), lambda i: (0, i, 0)),
            pl.BlockSpec((2, R, 16), lambda i: (0, i, 0)),
            pl.BlockSpec((2, R, 8), lambda i: (0, i, 0)),
            pl.BlockSpec((112, H), lambda i: (0, 0)),
            pl.BlockSpec((1, H), lambda i: (0, 0)),
        ],
        out_specs=pl.BlockSpec((2, R, 32), lambda i: (0, i, 0)),
        out_shape=jax.ShapeDtypeStruct((2, N, 32), _f32),
    )(nf, a0, agg_e, agg_t, wb0, b0r)

    a1 = _sc_segsum32(h1.reshape(NC * N, 32), src2, dst2d).reshape(2, N, 32)

    psums = pl.pallas_call(
        _tc_layer1,
        grid=(NBLK,),
        in_specs=[
            pl.BlockSpec((2, R, 32), lambda i: (0, i, 0)),
            pl.BlockSpec((2, R, 32), lambda i: (0, i, 0)),
            pl.BlockSpec((2, R, 16), lambda i: (0, i, 0)),
            pl.BlockSpec((2, R, 8), lambda i: (0, i, 0)),
            pl.BlockSpec((176, H), lambda i: (0, 0)),
            pl.BlockSpec((1, H), lambda i: (0, 0)),
        ],
        out_specs=pl.BlockSpec((NBLK, H), lambda i: (0, 0)),
        out_shape=jax.ShapeDtypeStruct((NBLK, H), _f32),
    )(h1, a1, agg_e, agg_t, wb1, b1r)

    out = pl.pallas_call(
        _tc_final,
        out_shape=jax.ShapeDtypeStruct((1, NCLS), _f32),
    )(psums, W_fc, bfcr)
    return out


# speedup vs baseline: 8.0705x; 1.0745x over previous
import functools

import jax
import jax.numpy as jnp
from jax import lax
from jax.experimental import pallas as pl
from jax.experimental.pallas import tpu as pltpu
from jax.experimental.pallas import tpu_sc as plsc

N = 50000
E = 800000
H = 64
NCLS = 10

NC = 2
NS = 16
LANES = 16

CH = 128
KPG = 8
GRP = CH * KPG
GROUPS = 49
EPT = GROUPS * GRP
EPAD = NS * EPT
NACC = 50176
ZPT = NACC // NS
DPT = N // NS

NLPAD = 204800
NLPT = NLPAD // (NC * NS)

R = 2000
NBLK = N // R

_f32 = jnp.float32
_i32 = jnp.int32

_mesh = plsc.VectorSubcoreMesh(
    core_axis_name="c", subcore_axis_name="s", num_cores=NC, num_subcores=NS)
_sc_params = pltpu.CompilerParams(use_tc_tiling_on_sc=False)


def _zero_fill(buf, rows, w):
    def zb(i, carry):
        for c in range(w // LANES):
            buf[i, pl.ds(c * LANES, LANES)] = jnp.zeros((LANES,), _f32)
        return carry
    lax.fori_loop(0, rows, zb, 0)


def _zero_acc(acc, buf, sid, grp):
    zb0 = sid * ZPT
    off = 0
    while off < ZPT:
        sz = min(grp, ZPT - off)
        pltpu.sync_copy(buf.at[pl.ds(0, sz)], acc.at[pl.ds(zb0 + off, sz)])
        off += sz


def _drain_acc(acc, out_hbm, cid, sid):
    db = sid * DPT
    ob = cid * N + db
    for off, sz in ((0, 1024), (1024, 1024), (2048, 1024), (3072, DPT - 3072)):
        pltpu.sync_copy(acc.at[pl.ds(db + off, sz)],
                        out_hbm.at[pl.ds(ob + off, sz)])


@functools.partial(
    pl.kernel,
    out_type=jax.ShapeDtypeStruct((NLPAD, 16), _f32),
    mesh=_mesh,
    compiler_params=_sc_params,
    scratch_types=[
        pltpu.VMEM((NLPT,), _i32),
        pltpu.VMEM((1280, 16), _f32),
        pltpu.SemaphoreType.DMA,
    ],
)
def _sc_node_embed(emb_hbm, nl_hbm, out_hbm, idxv, rows, sem):
    cid = lax.axis_index("c")
    sid = lax.axis_index("s")
    wid = sid * NC + cid
    base = wid * NLPT
    pltpu.sync_copy(nl_hbm.at[pl.ds(base, NLPT)], idxv)

    def gbody(g, carry):
        gb = g * 1280
        cps = []
        for k in range(10):
            cps.append(pltpu.async_copy(
                emb_hbm.at[idxv.at[pl.ds(gb + k * CH, CH)]],
                rows.at[pl.ds(k * CH, CH)], sem))
        for cp in cps:
            cp.wait()
        pltpu.sync_copy(rows, out_hbm.at[pl.ds(base + gb, 1280)])
        return carry
    lax.fori_loop(0, NLPT // 1280, gbody, 0)


def _make_segsum(w, kpg):
    grp = kpg * CH
    groups = EPT // grp

    @functools.partial(
        pl.kernel,
        out_type=jax.ShapeDtypeStruct((NC * N, w), _f32),
        mesh=_mesh,
        compiler_params=_sc_params,
        scratch_types=[
            pltpu.VMEM_SHARED((NACC, w), _f32),
            pltpu.VMEM((grp,), _i32),
            pltpu.VMEM((kpg, CH), _i32),
            pltpu.VMEM((grp, w), _f32),
            pltpu.SemaphoreType.DMA,
            pltpu.SemaphoreType.DMA,
        ],
    )
    def k(tbl_hbm, src_hbm, dst_hbm, out_hbm, acc, srcv, dstv, rows, sem, ssem):
        cid = lax.axis_index("c")
        sid = lax.axis_index("s")
        _zero_fill(rows, grp, w)
        _zero_acc(acc, rows, sid, grp)
        plsc.subcore_barrier()

        tb = sid * EPT
        itb = cid * EPAD + tb
        rtb = tb // CH

        def gbody(g, carry):
            gb = g * grp
            pltpu.sync_copy(src_hbm.at[pl.ds(itb + gb, grp)], srcv)
            pltpu.sync_copy(dst_hbm.at[pl.ds(rtb + g * kpg, kpg)], dstv)
            cps = []
            for k_ in range(kpg):
                cps.append(pltpu.async_copy(
                    tbl_hbm.at[srcv.at[pl.ds(k_ * CH, CH)]],
                    rows.at[pl.ds(k_ * CH, CH)], sem))
            scs = []
            for k_ in range(kpg):
                cps[k_].wait()
                scs.append(pltpu.async_copy(
                    rows.at[pl.ds(k_ * CH, CH)],
                    acc.at[dstv.at[k_]], ssem, add=True))
            for sc in scs:
                sc.wait()
            return carry
        lax.fori_loop(0, groups, gbody, 0)
        plsc.subcore_barrier()
        _drain_acc(acc, out_hbm, cid, sid)
    return k


_sc_segsum32 = _make_segsum(32, 4)


S16_KPG = 8
S16_GRP = S16_KPG * CH
S16_GROUPS = EPT // S16_GRP
S16_SPLIT = 25


@functools.partial(
    pl.kernel,
    out_type=(jax.ShapeDtypeStruct((NC * N, 16), _f32),
              jax.ShapeDtypeStruct((NC * N, 8), _f32)),
    mesh=_mesh,
    compiler_params=_sc_params,
    scratch_types=[
        pltpu.VMEM_SHARED((NACC, 16), _f32),
        pltpu.VMEM_SHARED((NACC, 8), _f32),
        pltpu.VMEM((S16_GRP,), _i32),
        pltpu.VMEM((S16_KPG, CH), _i32),
        pltpu.VMEM((S16_GRP, 16), _f32),
        pltpu.VMEM((S16_GRP, 8), _f32),
        pltpu.SemaphoreType.DMA,
        pltpu.SemaphoreType.DMA,
    ],
)
def _sc_segsum16et(tbl_hbm, src_hbm, dst_hbm, et_hbm, out_hbm, outt_hbm,
                   acc, acc_t, srcv, dstv, rows, etv, sem, ssem):
    cid = lax.axis_index("c")
    sid = lax.axis_index("s")
    _zero_fill(rows, S16_GRP, 16)
    _zero_acc(acc, rows, sid, S16_GRP)

    def z8(i, carry):
        etv[i, :] = jnp.zeros((8,), _f32)
        return carry
    lax.fori_loop(0, S16_GRP, z8, 0)
    _zero_acc(acc_t, etv, sid, S16_GRP)
    plsc.subcore_barrier()

    tb = sid * EPT
    itb = cid * EPAD + tb
    rtb = tb // CH

    def gbody(g, carry):
        gb = g * S16_GRP
        pltpu.sync_copy(src_hbm.at[pl.ds(itb + gb, S16_GRP)], srcv)
        pltpu.sync_copy(dst_hbm.at[pl.ds(rtb + g * S16_KPG, S16_KPG)], dstv)
        cps = []
        for k_ in range(S16_KPG):
            cps.append(pltpu.async_copy(
                tbl_hbm.at[srcv.at[pl.ds(k_ * CH, CH)]],
                rows.at[pl.ds(k_ * CH, CH)], sem))
        mine = jnp.where(cid == 0, g < S16_SPLIT, g >= S16_SPLIT)
        scs = []

        @pl.when(mine)
        def _():
            for k_ in range(S16_KPG):
                r0 = tb + gb + k_ * CH

                @pl.when(r0 < E)
                def _chunk(k_=k_, r0=r0):
                    pltpu.sync_copy(et_hbm.at[pl.ds(r0, CH)],
                                    etv.at[pl.ds(k_ * CH, CH)])
                    pltpu.sync_copy(etv.at[pl.ds(k_ * CH, CH)],
                                    acc_t.at[dstv.at[k_]], add=True)

        for k_ in range(S16_KPG):
            cps[k_].wait()
            scs.append(pltpu.async_copy(
                rows.at[pl.ds(k_ * CH, CH)],
                acc.at[dstv.at[k_]], ssem, add=True))
        for sc in scs:
            sc.wait()
        return carry
    lax.fori_loop(0, S16_GROUPS, gbody, 0)
    plsc.subcore_barrier()
    _drain_acc(acc, out_hbm, cid, sid)
    _drain_acc(acc_t, outt_hbm, cid, sid)


EA_KPG = 8
EA_GRP = EA_KPG * CH
EA_GROUPS = EPT // EA_GRP


@functools.partial(
    pl.kernel,
    out_type=jax.ShapeDtypeStruct((NC * N, 16), _f32),
    mesh=_mesh,
    compiler_params=_sc_params,
    scratch_types=[
        pltpu.VMEM_SHARED((NACC, 16), _f32),
        pltpu.VMEM((EA_GRP,), _i32),
        pltpu.VMEM((EA_KPG, CH), _i32),
        pltpu.VMEM((EA_GRP, 16), _f32),
        pltpu.SemaphoreType.DMA,
        pltpu.SemaphoreType.DMA,
    ],
)
def _sc_edge_aggs(emb_hbm, el_hbm, dst_hbm, oute_hbm,
                  acc_e, elv, dstv, erows, sem, ssem):
    cid = lax.axis_index("c")
    sid = lax.axis_index("s")
    _zero_fill(erows, EA_GRP, 16)
    _zero_acc(acc_e, erows, sid, EA_GRP)
    plsc.subcore_barrier()

    tb = sid * EPT
    itb = cid * EPAD + tb
    rtb = tb // CH

    def gbody(g, carry):
        gb = g * EA_GRP
        pltpu.sync_copy(el_hbm.at[pl.ds(itb + gb, EA_GRP)], elv)
        pltpu.sync_copy(dst_hbm.at[pl.ds(rtb + g * EA_KPG, EA_KPG)], dstv)
        cps = []
        for k_ in range(EA_KPG):
            cps.append(pltpu.async_copy(
                emb_hbm.at[elv.at[pl.ds(k_ * CH, CH)]],
                erows.at[pl.ds(k_ * CH, CH)], sem))
        scs = []
        for k_ in range(EA_KPG):
            cps[k_].wait()
            scs.append(pltpu.async_copy(
                erows.at[pl.ds(k_ * CH, CH)],
                acc_e.at[dstv.at[k_]], ssem, add=True))
        for sc in scs:
            sc.wait()
        return carry
    lax.fori_loop(0, EA_GROUPS, gbody, 0)
    plsc.subcore_barrier()
    _drain_acc(acc_e, oute_hbm, cid, sid)


def _tc_build_nf(nt_ref, ne_ref, out_ref):
    nt = nt_ref[...]
    ne = ne_ref[...].reshape(R, 4, 16)
    q0 = ne[:, 0, :]
    q1 = ne[:, 1, :]
    q2 = ne[:, 2, :]
    q3 = ne[:, 3, :]
    h0 = jnp.concatenate([nt, q0[:, 0:5], q1[:, 0:3]], axis=1)
    h1 = jnp.concatenate([q1[:, 3:5], q2[:, 0:5], q3[:, 0:5],
                          jnp.zeros((R, 4), _f32)], axis=1)
    out_ref[0] = h0
    out_ref[1] = h1


def _tc_layer0(nf_ref, a_ref, ae_ref, at_ref, wb_ref, b_ref, out_ref):
    x = jnp.concatenate([nf_ref[0], nf_ref[1], a_ref[0], a_ref[1],
                         ae_ref[0], ae_ref[1], at_ref[0], at_ref[1]], axis=1)
    h = jnp.dot(x, wb_ref[...], preferred_element_type=_f32) + b_ref[...]
    h = jnp.maximum(h, 0.0)
    out_ref[0] = h[:, :32]
    out_ref[1] = h[:, 32:]


def _tc_layer1(h_ref, a_ref, ae_ref, at_ref, wb_ref, b_ref, out_ref):
    x = jnp.concatenate([h_ref[0], h_ref[1], a_ref[0], a_ref[1],
                         ae_ref[0], ae_ref[1], at_ref[0], at_ref[1]], axis=1)
    h = jnp.dot(x, wb_ref[...], preferred_element_type=_f32) + b_ref[...]
    h = jnp.maximum(h, 0.0)
    out_ref[pl.ds(pl.program_id(0), 1), :] = jnp.sum(h, axis=0, keepdims=True)


def _tc_final(ps_ref, wfc_ref, bfc_ref, out_ref):
    s = jnp.sum(ps_ref[...], axis=0, keepdims=True)
    out_ref[...] = jnp.dot(s, wfc_ref[...], preferred_element_type=_f32) \
        + bfc_ref[...]


def _pad_rows(w, rows):
    return jnp.concatenate(
        [w, jnp.zeros((rows - w.shape[0], w.shape[1]), _f32)], axis=0)


def kernel(node_types, node_labels, edge_types, edge_labels, edge_index, emb,
           W_self0, W_src0, W_edge0, b0, W_self1, W_src1, W_edge1, b1,
           W_fc, b_fc):
    emb16 = jnp.zeros((emb.shape[0], 16), _f32).at[:, :5].set(emb)
    nl_flat = jnp.concatenate(
        [node_labels.reshape(-1).astype(_i32),
         jnp.zeros((NLPAD - 4 * N,), _i32)])

    src = edge_index[0].astype(_i32)
    dst = edge_index[1].astype(_i32)
    padn = EPAD - E
    srcp = jnp.concatenate([src, jnp.zeros((padn,), _i32)])
    src2 = jnp.concatenate([srcp, srcp + N])
    dstp = jnp.concatenate([dst, jnp.full((padn,), N, _i32)])
    dst2d = dstp.reshape(EPAD // CH, CH)
    el0 = edge_labels[:, 0].astype(_i32)
    el1 = edge_labels[:, 1].astype(_i32)
    el2 = jnp.concatenate([
        jnp.concatenate([el0, jnp.zeros((padn,), _i32)]),
        jnp.concatenate([el1, jnp.zeros((padn,), _i32)])])

    wb0 = jnp.concatenate([
        _pad_rows(W_self0, 32), _pad_rows(W_src0, 32),
        _pad_rows(W_edge0[0:5], 16), _pad_rows(W_edge0[5:10], 16),
        W_edge0[10:18], W_edge0[10:18]], axis=0)
    wb1 = jnp.concatenate([
        W_self1, W_src1,
        _pad_rows(W_edge1[0:5], 16), _pad_rows(W_edge1[5:10], 16),
        W_edge1[10:18], W_edge1[10:18]], axis=0)
    b0r = b0.reshape(1, H)
    b1r = b1.reshape(1, H)
    bfcr = b_fc.reshape(1, NCLS)

    ne_pad = _sc_node_embed(emb16, nl_flat)

    nf = pl.pallas_call(
        _tc_build_nf,
        grid=(NBLK,),
        in_specs=[
            pl.BlockSpec((R, 8), lambda i: (i, 0)),
            pl.BlockSpec((4 * R, 16), lambda i: (i, 0)),
        ],
        out_specs=pl.BlockSpec((2, R, 16), lambda i: (0, i, 0)),
        out_shape=jax.ShapeDtypeStruct((2, N, 16), _f32),
    )(node_types, ne_pad)

    a0, agg_t = _sc_segsum16et(nf.reshape(NC * N, 16), src2, dst2d, edge_types)
    agg_e = _sc_edge_aggs(emb16, el2, dst2d)
    a0 = a0.reshape(2, N, 16)
    agg_e = agg_e.reshape(2, N, 16)
    agg_t = agg_t.reshape(2, N, 8)

    h1 = pl.pallas_call(
        _tc_layer0,
        grid=(NBLK,),
        in_specs=[
            pl.BlockSpec((2, R, 16), lambda i: (0, i, 0)),
            pl.BlockSpec((2, R, 16), lambda i: (0, i, 0)),
            pl.BlockSpec((2, R, 16), lambda i: (0, i, 0)),
            pl.BlockSpec((2, R, 8), lambda i: (0, i, 0)),
            pl.BlockSpec((112, H), lambda i: (0, 0)),
            pl.BlockSpec((1, H), lambda i: (0, 0)),
        ],
        out_specs=pl.BlockSpec((2, R, 32), lambda i: (0, i, 0)),
        out_shape=jax.ShapeDtypeStruct((2, N, 32), _f32),
    )(nf, a0, agg_e, agg_t, wb0, b0r)

    a1 = _sc_segsum32(h1.reshape(NC * N, 32), src2, dst2d).reshape(2, N, 32)

    psums = pl.pallas_call(
        _tc_layer1,
        grid=(NBLK,),
        in_specs=[
            pl.BlockSpec((2, R, 32), lambda i: (0, i, 0)),
            pl.BlockSpec((2, R, 32), lambda i: (0, i, 0)),
            pl.BlockSpec((2, R, 16), lambda i: (0, i, 0)),
            pl.BlockSpec((2, R, 8), lambda i: (0, i, 0)),
            pl.BlockSpec((176, H), lambda i: (0, 0)),
            pl.BlockSpec((1, H), lambda i: (0, 0)),
        ],
        out_specs=pl.BlockSpec((NBLK, H), lambda i: (0, 0)),
        out_shape=jax.ShapeDtypeStruct((NBLK, H), _f32),
    )(h1, a1, agg_e, agg_t, wb1, b1r)

    out = pl.pallas_call(
        _tc_final,
        out_shape=jax.ShapeDtypeStruct((1, NCLS), _f32),
    )(psums, W_fc, bfcr)
    return out


# speedup vs baseline: 8.1212x; 1.0063x over previous
import functools

import jax
import jax.numpy as jnp
from jax import lax
from jax.experimental import pallas as pl
from jax.experimental.pallas import tpu as pltpu
from jax.experimental.pallas import tpu_sc as plsc

N = 50000
E = 800000
H = 64
NCLS = 10

NC = 2
NS = 16
LANES = 16

CH = 128
KPG = 8
GRP = CH * KPG
GROUPS = 49
EPT = GROUPS * GRP
EPAD = NS * EPT
NACC = 50176
ZPT = NACC // NS
DPT = N // NS

NLPAD = 204800
NLPT = NLPAD // (NC * NS)

R = 2000
NBLK = N // R

_f32 = jnp.float32
_i32 = jnp.int32

_mesh = plsc.VectorSubcoreMesh(
    core_axis_name="c", subcore_axis_name="s", num_cores=NC, num_subcores=NS)
_sc_params = pltpu.CompilerParams(use_tc_tiling_on_sc=False)


def _zero_fill(buf, rows, w):
    def zb(i, carry):
        for c in range(w // LANES):
            buf[i, pl.ds(c * LANES, LANES)] = jnp.zeros((LANES,), _f32)
        return carry
    lax.fori_loop(0, rows, zb, 0)


def _zero_acc(acc, buf, sid, grp):
    zb0 = sid * ZPT
    off = 0
    while off < ZPT:
        sz = min(grp, ZPT - off)
        pltpu.sync_copy(buf.at[pl.ds(0, sz)], acc.at[pl.ds(zb0 + off, sz)])
        off += sz


def _drain_acc(acc, out_hbm, cid, sid):
    db = sid * DPT
    ob = cid * N + db
    for off, sz in ((0, 1024), (1024, 1024), (2048, 1024), (3072, DPT - 3072)):
        pltpu.sync_copy(acc.at[pl.ds(db + off, sz)],
                        out_hbm.at[pl.ds(ob + off, sz)])


@functools.partial(
    pl.kernel,
    out_type=jax.ShapeDtypeStruct((NLPAD, 16), _f32),
    mesh=_mesh,
    compiler_params=_sc_params,
    scratch_types=[
        pltpu.VMEM((NLPT,), _i32),
        pltpu.VMEM((1280, 16), _f32),
        pltpu.SemaphoreType.DMA,
    ],
)
def _sc_node_embed(emb_hbm, nl_hbm, out_hbm, idxv, rows, sem):
    cid = lax.axis_index("c")
    sid = lax.axis_index("s")
    wid = sid * NC + cid
    base = wid * NLPT
    pltpu.sync_copy(nl_hbm.at[pl.ds(base, NLPT)], idxv)

    def gbody(g, carry):
        gb = g * 1280
        cps = []
        for k in range(10):
            cps.append(pltpu.async_copy(
                emb_hbm.at[idxv.at[pl.ds(gb + k * CH, CH)]],
                rows.at[pl.ds(k * CH, CH)], sem))
        for cp in cps:
            cp.wait()
        pltpu.sync_copy(rows, out_hbm.at[pl.ds(base + gb, 1280)])
        return carry
    lax.fori_loop(0, NLPT // 1280, gbody, 0)


def _make_segsum(w, kpg):
    grp = kpg * CH
    groups = EPT // grp

    @functools.partial(
        pl.kernel,
        out_type=jax.ShapeDtypeStruct((NC * N, w), _f32),
        mesh=_mesh,
        compiler_params=_sc_params,
        scratch_types=[
            pltpu.VMEM_SHARED((NACC, w), _f32),
            pltpu.VMEM((grp,), _i32),
            pltpu.VMEM((kpg, CH), _i32),
            pltpu.VMEM((grp, w), _f32),
            pltpu.SemaphoreType.DMA,
            pltpu.SemaphoreType.DMA,
        ],
    )
    def k(tbl_hbm, src_hbm, dst_hbm, out_hbm, acc, srcv, dstv, rows, sem, ssem):
        cid = lax.axis_index("c")
        sid = lax.axis_index("s")
        _zero_fill(rows, grp, w)
        _zero_acc(acc, rows, sid, grp)
        plsc.subcore_barrier()

        tb = sid * EPT
        itb = cid * EPAD + tb
        rtb = tb // CH

        def gbody(g, carry):
            gb = g * grp
            pltpu.sync_copy(src_hbm.at[pl.ds(itb + gb, grp)], srcv)
            pltpu.sync_copy(dst_hbm.at[pl.ds(rtb + g * kpg, kpg)], dstv)
            cps = []
            for k_ in range(kpg):
                cps.append(pltpu.async_copy(
                    tbl_hbm.at[srcv.at[pl.ds(k_ * CH, CH)]],
                    rows.at[pl.ds(k_ * CH, CH)], sem))
            scs = []
            for k_ in range(kpg):
                cps[k_].wait()
                scs.append(pltpu.async_copy(
                    rows.at[pl.ds(k_ * CH, CH)],
                    acc.at[dstv.at[k_]], ssem, add=True))
            for sc in scs:
                sc.wait()
            return carry
        lax.fori_loop(0, groups, gbody, 0)
        plsc.subcore_barrier()
        _drain_acc(acc, out_hbm, cid, sid)
    return k


_sc_segsum32 = _make_segsum(32, 4)


S16_KPG = 8
S16_GRP = S16_KPG * CH
S16_GROUPS = EPT // S16_GRP
S16_SPLIT = 25


@functools.partial(
    pl.kernel,
    out_type=(jax.ShapeDtypeStruct((NC * N, 16), _f32),
              jax.ShapeDtypeStruct((NC * N, 8), _f32)),
    mesh=_mesh,
    compiler_params=_sc_params,
    scratch_types=[
        pltpu.VMEM_SHARED((NACC, 16), _f32),
        pltpu.VMEM_SHARED((NACC, 8), _f32),
        pltpu.VMEM((S16_GRP,), _i32),
        pltpu.VMEM((S16_KPG, CH), _i32),
        pltpu.VMEM((S16_GRP, 16), _f32),
        pltpu.VMEM((S16_GRP, 8), _f32),
        pltpu.SemaphoreType.DMA,
        pltpu.SemaphoreType.DMA,
    ],
)
def _sc_segsum16et(tbl_hbm, src_hbm, dst_hbm, et_hbm, out_hbm, outt_hbm,
                   acc, acc_t, srcv, dstv, rows, etv, sem, ssem):
    cid = lax.axis_index("c")
    sid = lax.axis_index("s")
    _zero_fill(rows, S16_GRP, 16)
    _zero_acc(acc, rows, sid, S16_GRP)

    def z8(i, carry):
        etv[i, :] = jnp.zeros((8,), _f32)
        return carry
    lax.fori_loop(0, S16_GRP, z8, 0)
    _zero_acc(acc_t, etv, sid, S16_GRP)
    plsc.subcore_barrier()

    tb = sid * EPT
    itb = cid * EPAD + tb
    rtb = tb // CH

    def gbody(g, carry):
        gb = g * S16_GRP
        pltpu.sync_copy(src_hbm.at[pl.ds(itb + gb, S16_GRP)], srcv)
        pltpu.sync_copy(dst_hbm.at[pl.ds(rtb + g * S16_KPG, S16_KPG)], dstv)
        cps = []
        for k_ in range(S16_KPG):
            cps.append(pltpu.async_copy(
                tbl_hbm.at[srcv.at[pl.ds(k_ * CH, CH)]],
                rows.at[pl.ds(k_ * CH, CH)], sem))
        mine = jnp.where(cid == 0, g < S16_SPLIT, g >= S16_SPLIT)
        scs = []

        @pl.when(mine)
        def _():
            for k_ in range(S16_KPG):
                r0 = tb + gb + k_ * CH

                @pl.when(r0 < E)
                def _chunk(k_=k_, r0=r0):
                    pltpu.sync_copy(et_hbm.at[pl.ds(r0, CH)],
                                    etv.at[pl.ds(k_ * CH, CH)])
                    pltpu.sync_copy(etv.at[pl.ds(k_ * CH, CH)],
                                    acc_t.at[dstv.at[k_]], add=True)

        for k_ in range(S16_KPG):
            cps[k_].wait()
            scs.append(pltpu.async_copy(
                rows.at[pl.ds(k_ * CH, CH)],
                acc.at[dstv.at[k_]], ssem, add=True))
        for sc in scs:
            sc.wait()
        return carry
    lax.fori_loop(0, S16_GROUPS, gbody, 0)
    plsc.subcore_barrier()
    _drain_acc(acc, out_hbm, cid, sid)
    _drain_acc(acc_t, outt_hbm, cid, sid)


EA_KPG = 8
EA_GRP = EA_KPG * CH
EA_GROUPS = EPT // EA_GRP


@functools.partial(
    pl.kernel,
    out_type=jax.ShapeDtypeStruct((NC * N, 16), _f32),
    mesh=_mesh,
    compiler_params=_sc_params,
    scratch_types=[
        pltpu.VMEM_SHARED((NACC, 16), _f32),
        pltpu.VMEM((EA_GRP,), _i32),
        pltpu.VMEM((EA_KPG, CH), _i32),
        pltpu.VMEM((EA_GRP, 16), _f32),
        pltpu.SemaphoreType.DMA,
        pltpu.SemaphoreType.DMA,
    ],
)
def _sc_edge_aggs(emb_hbm, el_hbm, dst_hbm, oute_hbm,
                  acc_e, elv, dstv, erows, sem, ssem):
    cid = lax.axis_index("c")
    sid = lax.axis_index("s")
    _zero_fill(erows, EA_GRP, 16)
    _zero_acc(acc_e, erows, sid, EA_GRP)
    plsc.subcore_barrier()

    tb = sid * EPT
    itb = cid * EPAD + tb
    rtb = tb // CH

    def gbody(g, carry):
        gb = g * EA_GRP
        pltpu.sync_copy(el_hbm.at[pl.ds(itb + gb, EA_GRP)], elv)
        pltpu.sync_copy(dst_hbm.at[pl.ds(rtb + g * EA_KPG, EA_KPG)], dstv)
        cps = []
        for k_ in range(EA_KPG):
            cps.append(pltpu.async_copy(
                emb_hbm.at[elv.at[pl.ds(k_ * CH, CH)]],
                erows.at[pl.ds(k_ * CH, CH)], sem))
        scs = []
        for k_ in range(EA_KPG):
            cps[k_].wait()
            scs.append(pltpu.async_copy(
                erows.at[pl.ds(k_ * CH, CH)],
                acc_e.at[dstv.at[k_]], ssem, add=True))
        for sc in scs:
            sc.wait()
        return carry
    lax.fori_loop(0, EA_GROUPS, gbody, 0)
    plsc.subcore_barrier()
    _drain_acc(acc_e, oute_hbm, cid, sid)


def _tc_build_nf(nt_ref, ne_ref, out_ref):
    nt = nt_ref[...]
    ne = ne_ref[...].reshape(R, 4, 16)
    q0 = ne[:, 0, :]
    q1 = ne[:, 1, :]
    q2 = ne[:, 2, :]
    q3 = ne[:, 3, :]
    h0 = jnp.concatenate([nt, q0[:, 0:5], q1[:, 0:3]], axis=1)
    h1 = jnp.concatenate([q1[:, 3:5], q2[:, 0:5], q3[:, 0:5],
                          jnp.zeros((R, 4), _f32)], axis=1)
    out_ref[0] = h0
    out_ref[1] = h1


def _dot(x, w):
    return jnp.dot(x, w, preferred_element_type=_f32)


def _tc_layer0(nf_ref, a0l_ref, a0h_ref, ael_ref, aeh_ref, atl_ref, ath_ref,
               wb_ref, b_ref, out_ref):
    w = wb_ref[...]
    h = (_dot(nf_ref[0], w[0:16]) + _dot(nf_ref[1], w[16:32])
         + _dot(a0l_ref[...], w[32:48]) + _dot(a0h_ref[...], w[48:64])
         + _dot(ael_ref[...], w[64:80]) + _dot(aeh_ref[...], w[80:96])
         + _dot(atl_ref[...], w[96:104]) + _dot(ath_ref[...], w[104:112])
         + b_ref[...])
    h = jnp.maximum(h, 0.0)
    out_ref[0] = h[:, :32]
    out_ref[1] = h[:, 32:]


def _tc_layer1(h_ref, a1l_ref, a1h_ref, ael_ref, aeh_ref, atl_ref, ath_ref,
               wb_ref, b_ref, out_ref):
    w = wb_ref[...]
    h = (_dot(h_ref[0], w[0:32]) + _dot(h_ref[1], w[32:64])
         + _dot(a1l_ref[...], w[64:96]) + _dot(a1h_ref[...], w[96:128])
         + _dot(ael_ref[...], w[128:144]) + _dot(aeh_ref[...], w[144:160])
         + _dot(atl_ref[...], w[160:168]) + _dot(ath_ref[...], w[168:176])
         + b_ref[...])
    h = jnp.maximum(h, 0.0)
    out_ref[pl.ds(pl.program_id(0), 1), :] = jnp.sum(h, axis=0, keepdims=True)


def _tc_final(ps_ref, wfc_ref, bfc_ref, out_ref):
    s = jnp.sum(ps_ref[...], axis=0, keepdims=True)
    out_ref[...] = jnp.dot(s, wfc_ref[...], preferred_element_type=_f32) \
        + bfc_ref[...]


def _pad_rows(w, rows):
    return jnp.concatenate(
        [w, jnp.zeros((rows - w.shape[0], w.shape[1]), _f32)], axis=0)


def kernel(node_types, node_labels, edge_types, edge_labels, edge_index, emb,
           W_self0, W_src0, W_edge0, b0, W_self1, W_src1, W_edge1, b1,
           W_fc, b_fc):
    emb16 = jnp.zeros((emb.shape[0], 16), _f32).at[:, :5].set(emb)
    nl_flat = jnp.concatenate(
        [node_labels.reshape(-1).astype(_i32),
         jnp.zeros((NLPAD - 4 * N,), _i32)])

    src = edge_index[0].astype(_i32)
    dst = edge_index[1].astype(_i32)
    padn = EPAD - E
    srcp = jnp.concatenate([src, jnp.zeros((padn,), _i32)])
    src2 = jnp.concatenate([srcp, srcp + N])
    dstp = jnp.concatenate([dst, jnp.full((padn,), N, _i32)])
    dst2d = dstp.reshape(EPAD // CH, CH)
    el0 = edge_labels[:, 0].astype(_i32)
    el1 = edge_labels[:, 1].astype(_i32)
    el2 = jnp.concatenate([
        jnp.concatenate([el0, jnp.zeros((padn,), _i32)]),
        jnp.concatenate([el1, jnp.zeros((padn,), _i32)])])

    wb0 = jnp.concatenate([
        _pad_rows(W_self0, 32), _pad_rows(W_src0, 32),
        _pad_rows(W_edge0[0:5], 16), _pad_rows(W_edge0[5:10], 16),
        W_edge0[10:18], W_edge0[10:18]], axis=0)
    wb1 = jnp.concatenate([
        W_self1, W_src1,
        _pad_rows(W_edge1[0:5], 16), _pad_rows(W_edge1[5:10], 16),
        W_edge1[10:18], W_edge1[10:18]], axis=0)
    b0r = b0.reshape(1, H)
    b1r = b1.reshape(1, H)
    bfcr = b_fc.reshape(1, NCLS)

    ne_pad = _sc_node_embed(emb16, nl_flat)

    nf = pl.pallas_call(
        _tc_build_nf,
        grid=(NBLK,),
        in_specs=[
            pl.BlockSpec((R, 8), lambda i: (i, 0)),
            pl.BlockSpec((4 * R, 16), lambda i: (i, 0)),
        ],
        out_specs=pl.BlockSpec((2, R, 16), lambda i: (0, i, 0)),
        out_shape=jax.ShapeDtypeStruct((2, N, 16), _f32),
    )(node_types, ne_pad)

    a0, agg_t = _sc_segsum16et(nf.reshape(NC * N, 16), src2, dst2d, edge_types)
    agg_e = _sc_edge_aggs(emb16, el2, dst2d)

    lo = lambda i: (i, 0)
    hi = lambda i: (i + NBLK, 0)

    h1 = pl.pallas_call(
        _tc_layer0,
        grid=(NBLK,),
        in_specs=[
            pl.BlockSpec((2, R, 16), lambda i: (0, i, 0)),
            pl.BlockSpec((R, 16), lo), pl.BlockSpec((R, 16), hi),
            pl.BlockSpec((R, 16), lo), pl.BlockSpec((R, 16), hi),
            pl.BlockSpec((R, 8), lo), pl.BlockSpec((R, 8), hi),
            pl.BlockSpec((112, H), lambda i: (0, 0)),
            pl.BlockSpec((1, H), lambda i: (0, 0)),
        ],
        out_specs=pl.BlockSpec((2, R, 32), lambda i: (0, i, 0)),
        out_shape=jax.ShapeDtypeStruct((2, N, 32), _f32),
    )(nf, a0, a0, agg_e, agg_e, agg_t, agg_t, wb0, b0r)

    a1 = _sc_segsum32(h1.reshape(NC * N, 32), src2, dst2d)

    psums = pl.pallas_call(
        _tc_layer1,
        grid=(NBLK,),
        in_specs=[
            pl.BlockSpec((2, R, 32), lambda i: (0, i, 0)),
            pl.BlockSpec((R, 32), lo), pl.BlockSpec((R, 32), hi),
            pl.BlockSpec((R, 16), lo), pl.BlockSpec((R, 16), hi),
            pl.BlockSpec((R, 8), lo), pl.BlockSpec((R, 8), hi),
            pl.BlockSpec((176, H), lambda i: (0, 0)),
            pl.BlockSpec((1, H), lambda i: (0, 0)),
        ],
        out_specs=pl.BlockSpec((NBLK, H), lambda i: (0, 0)),
        out_shape=jax.ShapeDtypeStruct((NBLK, H), _f32),
    )(h1, a1, a1, agg_e, agg_e, agg_t, agg_t, wb1, b1r)

    out = pl.pallas_call(
        _tc_final,
        out_shape=jax.ShapeDtypeStruct((1, NCLS), _f32),
    )(psums, W_fc, bfcr)
    return out
